# Initial kernel scaffold; baseline (speedup 1.0000x reference)
import jax
import jax.numpy as jnp
from jax.experimental import pallas as pl


def kernel(context, response):
    raise NotImplementedError("write your pallas kernel here")


# speedup vs baseline: 25.1619x; 25.1619x over previous
import jax
import jax.numpy as jnp
from jax.experimental import pallas as pl
from jax.experimental.pallas import tpu as pltpu

_B = 4096
_D = 128
_BM = 256
_N_NEG = 8
_MARGIN = 0.2


def _rownorm(x):
    s = x * x
    a = s[:, 0:8]
    for m in range(1, 16):
        a = a + s[:, 8 * m:8 * m + 8]
    b = a[:, 4:8] + a[:, 0:4]
    c = b[:, 2:4] + b[:, 0:2]
    d = c[:, 1:2] + c[:, 0:1]
    return jnp.sqrt(d)


def _body(c_ref, r_ref, idx_ref, w_ref, rn_ref):
    i = pl.program_id(0)

    @pl.when(i == 0)
    def _():
        r = r_ref[...]
        rn_ref[...] = r / jnp.maximum(_rownorm(r), 1e-12)

    c = c_ref[...]
    cn = c / jnp.maximum(_rownorm(c), 1e-12)
    rn = rn_ref[...]
    sim = jax.lax.dot_general(cn, rn, (((1,), (1,)), ((), ())),
                              preferred_element_type=jnp.float32)

    col = jax.lax.broadcasted_iota(jnp.int32, (_BM, _B), 1)
    row_g = jax.lax.broadcasted_iota(jnp.int32, (_BM, _B), 0) + i * _BM
    d = jnp.sum(jnp.where(col == row_g, sim, 0.0), axis=1, keepdims=True)
    diff = d - sim
    work = jnp.where(diff > 0.0, sim, jnp.float32(-100.0))
    work = jnp.where(diff <= _MARGIN, work, jnp.float32(-10.0))

    vals, inds = [], []
    for _ in range(_N_NEG):
        m = jnp.max(work, axis=1, keepdims=True)
        sel = jnp.where(work == m, col, jnp.int32(_B))
        j = jnp.min(sel, axis=1, keepdims=True)
        vals.append(m)
        inds.append(j)
        work = jnp.where(col == j, jnp.float32(-1e30), work)

    v = jnp.concatenate(vals, axis=1)
    e = jnp.exp(v - v[:, 0:1])
    w_ref[...] = e / jnp.sum(e, axis=1, keepdims=True)
    idx_ref[...] = jnp.concatenate(inds, axis=1)


def kernel(context, response):
    grid = _B // _BM
    idx, w = pl.pallas_call(
        _body,
        grid=(grid,),
        in_specs=[
            pl.BlockSpec((_BM, _D), lambda i: (i, 0)),
            pl.BlockSpec((_B, _D), lambda i: (0, 0)),
        ],
        out_specs=[
            pl.BlockSpec((_BM, _N_NEG), lambda i: (i, 0)),
            pl.BlockSpec((_BM, _N_NEG), lambda i: (i, 0)),
        ],
        out_shape=[
            jax.ShapeDtypeStruct((_B, _N_NEG), jnp.int32),
            jax.ShapeDtypeStruct((_B, _N_NEG), jnp.float32),
        ],
        scratch_shapes=[pltpu.VMEM((_B, _D), jnp.float32)],
    )(context, response)
    return idx, w


# speedup vs baseline: 33.4908x; 1.3310x over previous
import jax
import jax.numpy as jnp
from jax.experimental import pallas as pl

_B = 4096
_D = 128
_BM = 256
_N_NEG = 8
_MARGIN = 0.2


def _norm_one(x):
    xt = x.T
    s = xt * xt
    a = s[0:8]
    for m in range(1, 16):
        a = a + s[8 * m:8 * m + 8]
    b = a[4:8] + a[0:4]
    c = b[2:4] + b[0:2]
    d = c[1:2] + c[0:1]
    n = jnp.maximum(jnp.sqrt(d), 1e-12)
    return (xt / n).T


def _norm_body(c_ref, r_ref, cn_ref, rn_ref):
    cn_ref[...] = _norm_one(c_ref[...])
    rn_ref[...] = _norm_one(r_ref[...])


def _body(cn_ref, rn_ref, idx_ref, w_ref):
    i = pl.program_id(0)
    cn = cn_ref[...]
    rn = rn_ref[...]
    sim = jax.lax.dot_general(cn, rn, (((1,), (1,)), ((), ())),
                              preferred_element_type=jnp.float32)

    row_g = jax.lax.broadcasted_iota(jnp.int32, (_BM, _B), 0) + i * _BM
    coli = jax.lax.broadcasted_iota(jnp.int32, (_BM, _B), 1)
    col = coli.astype(jnp.float32)
    d = jnp.sum(jnp.where(coli == row_g, sim, 0.0), axis=1, keepdims=True)
    diff = d - sim
    work = jnp.where(diff > 0.0, sim, jnp.float32(-100.0))
    work = jnp.where(diff <= _MARGIN, work, jnp.float32(-10.0))

    vals, inds = [], []
    for _ in range(_N_NEG):
        m = jnp.max(work, axis=1, keepdims=True)
        z = jnp.where(work == m, -col, jnp.float32(-3e38))
        j = -jnp.max(z, axis=1, keepdims=True)
        vals.append(m)
        inds.append(j)
        work = jnp.where(col == j, jnp.float32(-1e30), work)

    v = jnp.concatenate(vals, axis=1)
    e = jnp.exp(v - v[:, 0:1])
    w_ref[...] = e / jnp.sum(e, axis=1, keepdims=True)
    idx_ref[...] = jnp.concatenate(inds, axis=1).astype(jnp.int32)


def kernel(context, response):
    cn, rn = pl.pallas_call(
        _norm_body,
        out_shape=[
            jax.ShapeDtypeStruct((_B, _D), jnp.float32),
            jax.ShapeDtypeStruct((_B, _D), jnp.float32),
        ],
    )(context, response)

    grid = _B // _BM
    idx, w = pl.pallas_call(
        _body,
        grid=(grid,),
        in_specs=[
            pl.BlockSpec((_BM, _D), lambda i: (i, 0)),
            pl.BlockSpec((_B, _D), lambda i: (0, 0)),
        ],
        out_specs=[
            pl.BlockSpec((_BM, _N_NEG), lambda i: (i, 0)),
            pl.BlockSpec((_BM, _N_NEG), lambda i: (i, 0)),
        ],
        out_shape=[
            jax.ShapeDtypeStruct((_B, _N_NEG), jnp.int32),
            jax.ShapeDtypeStruct((_B, _N_NEG), jnp.float32),
        ],
    )(cn, rn)
    return idx, w


# speedup vs baseline: 35.0711x; 1.0472x over previous
import jax
import jax.numpy as jnp
from jax.experimental import pallas as pl
from jax.experimental.pallas import tpu as pltpu

_B = 4096
_D = 128
_BM = 512
_N_NEG = 8
_MARGIN = 0.2


def _norm_one(x):
    xt = x.T
    s = xt * xt
    a = s[0:8]
    for m in range(1, 16):
        a = a + s[8 * m:8 * m + 8]
    b = a[4:8] + a[0:4]
    c = b[2:4] + b[0:2]
    d = c[1:2] + c[0:1]
    n = jnp.maximum(jnp.sqrt(d), 1e-12)
    return (xt / n).T


def _body(c_ref, r_ref, idx_ref, w_ref, rn_ref):
    i = pl.program_id(0)

    @pl.when(i == 0)
    def _():
        rn_ref[...] = _norm_one(r_ref[...])

    cn = _norm_one(c_ref[...])
    rn = rn_ref[...]
    sim = jax.lax.dot_general(cn, rn, (((1,), (1,)), ((), ())),
                              preferred_element_type=jnp.float32)

    row_g = jax.lax.broadcasted_iota(jnp.int32, (_BM, _B), 0) + i * _BM
    coli = jax.lax.broadcasted_iota(jnp.int32, (_BM, _B), 1)
    col = coli.astype(jnp.float32)
    d = jnp.sum(jnp.where(coli == row_g, sim, 0.0), axis=1, keepdims=True)
    diff = d - sim
    work = jnp.where(diff > 0.0, sim, jnp.float32(-100.0))
    work = jnp.where(diff <= _MARGIN, work, jnp.float32(-10.0))

    vals, inds = [], []
    for _ in range(_N_NEG):
        m = jnp.max(work, axis=1, keepdims=True)
        z = jnp.where(work == m, -col, jnp.float32(-3e38))
        j = -jnp.max(z, axis=1, keepdims=True)
        vals.append(m)
        inds.append(j)
        work = jnp.where(col == j, jnp.float32(-1e30), work)

    v = jnp.concatenate(vals, axis=1)
    e = jnp.exp(v - v[:, 0:1])
    w_ref[...] = e / jnp.sum(e, axis=1, keepdims=True)
    idx_ref[...] = jnp.concatenate(inds, axis=1).astype(jnp.int32)


def kernel(context, response):
    grid = _B // _BM
    idx, w = pl.pallas_call(
        _body,
        grid=(grid,),
        in_specs=[
            pl.BlockSpec((_BM, _D), lambda i: (i, 0)),
            pl.BlockSpec((_B, _D), lambda i: (0, 0)),
        ],
        out_specs=[
            pl.BlockSpec((_BM, _N_NEG), lambda i: (i, 0)),
            pl.BlockSpec((_BM, _N_NEG), lambda i: (i, 0)),
        ],
        out_shape=[
            jax.ShapeDtypeStruct((_B, _N_NEG), jnp.int32),
            jax.ShapeDtypeStruct((_B, _N_NEG), jnp.float32),
        ],
        scratch_shapes=[pltpu.VMEM((_B, _D), jnp.float32)],
    )(context, response)
    return idx, w
